# baseline (speedup 1.0000x reference)
import functools

import jax
import jax.numpy as jnp
from jax import lax
from jax.experimental import pallas as pl
from jax.experimental.pallas import tpu as pltpu
from jax.experimental.pallas import tpu_sc as plsc

N = 10000
E = 320000
D = 128

NC = 2
NS = 16
NW = NC * NS
CH = 128
R2D = 2560
EPAD = R2D * CH
RPC = R2D // NW
CHS = 128
IBLK = 4
NBLK = EPAD // (CHS * IBLK * NW)
NPAD = 10240
RPT = NPAD // NS
DPT = NPAD // NS

_sc_mesh = plsc.VectorSubcoreMesh(core_axis_name="c", subcore_axis_name="s")


@functools.partial(
    pl.kernel,
    out_type=[jax.ShapeDtypeStruct((NC * NPAD,), jnp.float32)] * 4,
    mesh=_sc_mesh,
    scratch_types=[
        pltpu.VMEM((RPC, CH), jnp.int32),
        pltpu.VMEM((RPC, CH), jnp.int32),
        pltpu.VMEM((RPC, CH), jnp.int32),
        pltpu.VMEM((RPC, CH), jnp.int32),
        pltpu.VMEM((CH,), jnp.float32),
        pltpu.VMEM_SHARED((NPAD,), jnp.float32),
        pltpu.VMEM_SHARED((NPAD,), jnp.float32),
        pltpu.VMEM_SHARED((NPAD,), jnp.float32),
        pltpu.VMEM_SHARED((NPAD,), jnp.float32),
        pltpu.SemaphoreType.DMA,
        pltpu.SemaphoreType.DMA,
        pltpu.SemaphoreType.DMA,
        pltpu.SemaphoreType.DMA,
    ],
)
def _sc_degrees(s1_hbm, d1_hbm, s2_hbm, d2_hbm, ones_hbm, zeros_hbm,
                o1_hbm, i1_hbm, o2_hbm, i2_hbm,
                s1_v, d1_v, s2_v, d2_v, ones_v,
                o1_sh, i1_sh, o2_sh, i2_sh,
                sem0, sem1, sem2, sem3):
    cid = lax.axis_index("c")
    sid = lax.axis_index("s")
    wid = cid * NS + sid
    idx_v = (s1_v, d1_v, s2_v, d2_v)
    idx_hbm = (s1_hbm, d1_hbm, s2_hbm, d2_hbm)
    sh = (o1_sh, i1_sh, o2_sh, i2_sh)
    out = (o1_hbm, i1_hbm, o2_hbm, i2_hbm)
    sems = (sem0, sem1, sem2, sem3)

    pltpu.sync_copy(ones_hbm, ones_v)
    for k in range(4):
        pltpu.sync_copy(idx_hbm[k].at[pl.ds(wid * RPC, RPC)], idx_v[k])
        pltpu.sync_copy(zeros_hbm, sh[k].at[pl.ds(sid * DPT, DPT)])
    plsc.subcore_barrier()

    for k in range(4):
        pltpu.async_copy(ones_v, sh[k].at[idx_v[k].at[0]], sems[k], add=True)

    def body(j, carry):
        for k in range(4):
            pltpu.make_async_copy(ones_v, sh[k].at[idx_v[k].at[j - 1]],
                                  sems[k]).wait()
            pltpu.async_copy(ones_v, sh[k].at[idx_v[k].at[j]], sems[k],
                             add=True)
        return carry

    lax.fori_loop(1, RPC, body, 0)
    for k in range(4):
        pltpu.make_async_copy(ones_v, sh[k].at[idx_v[k].at[RPC - 1]],
                              sems[k]).wait()
    plsc.subcore_barrier()
    for k in range(4):
        pltpu.sync_copy(sh[k].at[pl.ds(sid * DPT, DPT)],
                        out[k].at[pl.ds(cid * NPAD + sid * DPT, DPT)])


@functools.partial(
    pl.kernel,
    out_type=jax.ShapeDtypeStruct((NC, NPAD, D), jnp.float32),
    mesh=_sc_mesh,
    scratch_types=[
        pltpu.VMEM((2 * IBLK, CHS), jnp.int32),
        pltpu.VMEM((CHS, D), jnp.float32),
        pltpu.VMEM((CHS, D), jnp.float32),
        pltpu.VMEM_SHARED((NPAD, D), jnp.float32),
        pltpu.SemaphoreType.DMA,
        pltpu.SemaphoreType.DMA,
    ],
)
def _sc_spmm(h_hbm, ib_hbm, zrows_hbm, out_hbm,
             ib_v, rows_a, rows_b, acc_sh, sem_a, sem_b):
    cid = lax.axis_index("c")
    sid = lax.axis_index("s")
    wid = cid * NS + sid

    pltpu.sync_copy(zrows_hbm, acc_sh.at[pl.ds(sid * RPT, RPT)])
    plsc.subcore_barrier()

    def body(t, carry):
        pltpu.sync_copy(ib_hbm.at[pl.ds((wid * NBLK + t) * 2 * IBLK,
                                        2 * IBLK)], ib_v)
        for i in range(IBLK):
            pltpu.async_copy(h_hbm.at[ib_v.at[i]], rows_a, sem_a).wait()
            pltpu.sync_copy(rows_a, acc_sh.at[ib_v.at[IBLK + i]], add=True)
        return carry

    lax.fori_loop(0, NBLK, body, 0)
    plsc.subcore_barrier()
    pltpu.sync_copy(acc_sh.at[pl.ds(sid * RPT, RPT)],
                    out_hbm.at[cid, pl.ds(sid * RPT, RPT)])


_BN = 2000


def _prep_body(feat_ref, doa_ref, dob_ref, dia_ref, dib_ref,
               h0_ref, ns_ref, nd_ref):
    dego = doa_ref[...] + dob_ref[...]
    degi = dia_ref[...] + dib_ref[...]
    ns = lax.rsqrt(jnp.where(dego > 0, dego, 1.0))
    nd = lax.rsqrt(jnp.where(degi > 0, degi, 1.0))
    ns_ref[...] = ns
    nd_ref[...] = nd
    h0_ref[...] = feat_ref[...] * ns


def _tc_prep(feat, doa, dob, dia, dib):
    grid = (N // _BN,)
    row = pl.BlockSpec((_BN, D), lambda i: (i, 0))
    col = pl.BlockSpec((_BN, 1), lambda i: (i, 0))
    return pl.pallas_call(
        _prep_body,
        grid=grid,
        in_specs=[row, col, col, col, col],
        out_specs=[row, col, col],
        out_shape=[
            jax.ShapeDtypeStruct((N, D), jnp.float32),
            jax.ShapeDtypeStruct((N, 1), jnp.float32),
            jax.ShapeDtypeStruct((N, 1), jnp.float32),
        ],
    )(feat, doa, dob, dia, dib)


def _mm_body(aa_ref, ab_ref, nd_ref, so_ref, w_ref, b_ref, y_ref):
    x = (aa_ref[0] + ab_ref[0]) * nd_ref[...]
    y = jnp.dot(x, w_ref[...], preferred_element_type=jnp.float32)
    y = jnp.maximum(y + b_ref[...], 0.0)
    y_ref[...] = y * so_ref[...]


def _tc_mm(aggp, nd, so, w, b):
    grid = (N // _BN,)
    part_a = pl.BlockSpec((1, _BN, D), lambda i: (0, i, 0))
    part_b = pl.BlockSpec((1, _BN, D), lambda i: (1, i, 0))
    row = pl.BlockSpec((_BN, D), lambda i: (i, 0))
    col = pl.BlockSpec((_BN, 1), lambda i: (i, 0))
    full = pl.BlockSpec((D, D), lambda i: (0, 0))
    vec = pl.BlockSpec((1, D), lambda i: (0, 0))
    return pl.pallas_call(
        _mm_body,
        grid=grid,
        in_specs=[part_a, part_b, col, col, full, vec],
        out_specs=row,
        out_shape=jax.ShapeDtypeStruct((N, D), jnp.float32),
    )(aggp, aggp, nd, so, w, b)


def _mlp_body(c_ref, w1_ref, b1_ref, w2_ref, b2_ref, y_ref):
    t = jnp.dot(c_ref[...], w1_ref[...], preferred_element_type=jnp.float32)
    t = jnp.maximum(t + b1_ref[...], 0.0)
    y = jnp.dot(t, w2_ref[...], preferred_element_type=jnp.float32)
    y_ref[...] = jnp.maximum(y + b2_ref[...], 0.0)


def _tc_mlp(c, w1, b1, w2, b2):
    grid = (N // _BN,)
    row = pl.BlockSpec((_BN, D), lambda i: (i, 0))
    full = pl.BlockSpec((D, D), lambda i: (0, 0))
    vec = pl.BlockSpec((1, D), lambda i: (0, 0))
    return pl.pallas_call(
        _mlp_body,
        grid=grid,
        in_specs=[row, full, vec, full, vec],
        out_specs=row,
        out_shape=jax.ShapeDtypeStruct((N, D), jnp.float32),
    )(c, w1, b1, w2, b2)


def _pad_edges(idx, pad_value):
    pad = jnp.full((EPAD - E,), pad_value, jnp.int32)
    return jnp.concatenate([idx, pad])


def _mk_iblocks(src_p, dst_p):
    s = src_p.reshape(-1, IBLK, CHS)
    d = dst_p.reshape(-1, IBLK, CHS)
    return jnp.concatenate([s, d], axis=1).reshape(-1, CHS)


def _gcn_encode(feat, iblocks, dego, degi, W1, b1, W2, b2, zrows):
    doa = dego[:N].reshape(N, 1)
    dob = dego[NPAD:NPAD + N].reshape(N, 1)
    dia = degi[:N].reshape(N, 1)
    dib = degi[NPAD:NPAD + N].reshape(N, 1)

    h0, ns, nd = _tc_prep(feat, doa, dob, dia, dib)

    agg1 = _sc_spmm(h0, iblocks, zrows)
    h1 = _tc_mm(agg1, nd, ns, W1, b1.reshape(1, D))

    agg2 = _sc_spmm(h1, iblocks, zrows)
    ones_n = jnp.ones((N, 1), jnp.float32)
    c = _tc_mm(agg2, nd, ones_n, W2, b2.reshape(1, D))
    return c


def kernel(feat1, feat2, edge_index1, edge_index2, W1, b1, W2, b2,
           Wm1, bm1, Wm2, bm2):
    s1, d1 = edge_index1[0], edge_index1[1]
    s2, d2 = edge_index2[0], edge_index2[1]
    s1_dg = _pad_edges(s1, N).reshape(R2D, CH)
    s2_dg = _pad_edges(s2, N).reshape(R2D, CH)
    d1_p = _pad_edges(d1, N)
    d2_p = _pad_edges(d2, N)
    ib1 = _mk_iblocks(_pad_edges(s1, 0), d1_p)
    ib2 = _mk_iblocks(_pad_edges(s2, 0), d2_p)

    ones_c = jnp.ones((CH,), jnp.float32)
    zeros_deg = jnp.zeros((DPT,), jnp.float32)
    zrows = jnp.zeros((RPT, D), jnp.float32)

    dego1, degi1, dego2, degi2 = _sc_degrees(
        s1_dg, d1_p.reshape(R2D, CH), s2_dg, d2_p.reshape(R2D, CH),
        ones_c, zeros_deg)

    c1 = _gcn_encode(feat1, ib1, dego1, degi1, W1, b1, W2, b2, zrows)
    c2 = _gcn_encode(feat2, ib2, dego2, degi2, W1, b1, W2, b2, zrows)

    x21 = _tc_mlp(c1, Wm1, bm1.reshape(1, D), Wm2, bm2.reshape(1, D))
    x22 = _tc_mlp(c2, Wm1, bm1.reshape(1, D), Wm2, bm2.reshape(1, D))
    return (c1, c2, x21, x22)


# speedup vs baseline: 1.6357x; 1.6357x over previous
import functools

import jax
import jax.numpy as jnp
from jax import lax
from jax.experimental import pallas as pl
from jax.experimental.pallas import tpu as pltpu
from jax.experimental.pallas import tpu_sc as plsc

N = 10000
E = 320000
D = 128

NC = 2
NS = 16
NW = NC * NS
CH = 128
R2D = 2560
EPAD = R2D * CH
RPC = R2D // NW
CHS = 80
ESW = E // NS
NCHS = ESW // CHS
NPAD = 10240
RPT = NPAD // NS
DPT = NPAD // NS

_sc_mesh = plsc.VectorSubcoreMesh(core_axis_name="c", subcore_axis_name="s")


@functools.partial(
    pl.kernel,
    out_type=[jax.ShapeDtypeStruct((NC * NPAD,), jnp.float32)] * 4,
    mesh=_sc_mesh,
    scratch_types=[
        pltpu.VMEM((RPC, CH), jnp.int32),
        pltpu.VMEM((RPC, CH), jnp.int32),
        pltpu.VMEM((RPC, CH), jnp.int32),
        pltpu.VMEM((RPC, CH), jnp.int32),
        pltpu.VMEM((CH,), jnp.float32),
        pltpu.VMEM_SHARED((NPAD,), jnp.float32),
        pltpu.VMEM_SHARED((NPAD,), jnp.float32),
        pltpu.VMEM_SHARED((NPAD,), jnp.float32),
        pltpu.VMEM_SHARED((NPAD,), jnp.float32),
        pltpu.SemaphoreType.DMA,
        pltpu.SemaphoreType.DMA,
        pltpu.SemaphoreType.DMA,
        pltpu.SemaphoreType.DMA,
    ],
)
def _sc_degrees(s1_hbm, d1_hbm, s2_hbm, d2_hbm, ones_hbm, zeros_hbm,
                o1_hbm, i1_hbm, o2_hbm, i2_hbm,
                s1_v, d1_v, s2_v, d2_v, ones_v,
                o1_sh, i1_sh, o2_sh, i2_sh,
                sem0, sem1, sem2, sem3):
    cid = lax.axis_index("c")
    sid = lax.axis_index("s")
    wid = cid * NS + sid
    idx_v = (s1_v, d1_v, s2_v, d2_v)
    idx_hbm = (s1_hbm, d1_hbm, s2_hbm, d2_hbm)
    sh = (o1_sh, i1_sh, o2_sh, i2_sh)
    out = (o1_hbm, i1_hbm, o2_hbm, i2_hbm)
    sems = (sem0, sem1, sem2, sem3)

    pltpu.sync_copy(ones_hbm, ones_v)
    for k in range(4):
        pltpu.sync_copy(idx_hbm[k].at[pl.ds(wid * RPC, RPC)], idx_v[k])
        pltpu.sync_copy(zeros_hbm, sh[k].at[pl.ds(sid * DPT, DPT)])
    plsc.subcore_barrier()

    for k in range(4):
        pltpu.async_copy(ones_v, sh[k].at[idx_v[k].at[0]], sems[k], add=True)

    def body(j, carry):
        for k in range(4):
            pltpu.make_async_copy(ones_v, sh[k].at[idx_v[k].at[j - 1]],
                                  sems[k]).wait()
            pltpu.async_copy(ones_v, sh[k].at[idx_v[k].at[j]], sems[k],
                             add=True)
        return carry

    lax.fori_loop(1, RPC, body, 0)
    for k in range(4):
        pltpu.make_async_copy(ones_v, sh[k].at[idx_v[k].at[RPC - 1]],
                              sems[k]).wait()
    plsc.subcore_barrier()
    for k in range(4):
        pltpu.sync_copy(sh[k].at[pl.ds(sid * DPT, DPT)],
                        out[k].at[pl.ds(cid * NPAD + sid * DPT, DPT)])


@functools.partial(
    pl.kernel,
    out_type=jax.ShapeDtypeStruct((NC, NPAD, D), jnp.float32),
    mesh=_sc_mesh,
    scratch_types=[
        pltpu.VMEM((CHS,), jnp.int32),
        pltpu.VMEM((CHS,), jnp.int32),
        pltpu.VMEM((CHS, D), jnp.float32),
        pltpu.VMEM_SHARED((NPAD, D), jnp.float32),
        pltpu.SemaphoreType.DMA,
    ],
)
def _sc_spmm(h_hbm, src_hbm, dst_hbm, zrows_hbm, out_hbm,
             sidx_v, didx_v, rows_v, acc_sh, sem):
    cid = lax.axis_index("c")
    sid = lax.axis_index("s")
    wid = cid * NS + sid
    base = wid * ESW

    pltpu.sync_copy(zrows_hbm, acc_sh.at[pl.ds(sid * RPT, RPT)])
    plsc.subcore_barrier()

    def body(i, carry):
        off = base + i * CHS
        pltpu.sync_copy(src_hbm.at[pl.ds(off, CHS)], sidx_v)
        pltpu.sync_copy(dst_hbm.at[pl.ds(off, CHS)], didx_v)
        pltpu.async_copy(h_hbm.at[sidx_v], rows_v, sem).wait()
        pltpu.sync_copy(rows_v, acc_sh.at[didx_v], add=True)
        return carry

    lax.fori_loop(0, NCHS, body, 0)
    plsc.subcore_barrier()
    pltpu.sync_copy(acc_sh.at[pl.ds(sid * RPT, RPT)],
                    out_hbm.at[cid, pl.ds(sid * RPT, RPT)])


_BN = 2000
_G = N // _BN


def _prep_body(feat_ref, do_ref, di_ref, h0_ref, ns_ref, nd_ref):
    dego = do_ref[0]
    degi = di_ref[0]
    ns = lax.rsqrt(jnp.where(dego > 0, dego, 1.0))
    nd = lax.rsqrt(jnp.where(degi > 0, degi, 1.0))
    ns_ref[0] = ns
    nd_ref[0] = nd
    h0_ref[0] = feat_ref[0] * ns


def _tc_prep(feat_cat, do_cat, di_cat):
    grid = (2 * _G,)
    row = pl.BlockSpec((1, _BN, D), lambda i: (i // _G, i % _G, 0))
    col = pl.BlockSpec((1, _BN, 1), lambda i: (i // _G, i % _G, 0))
    return pl.pallas_call(
        _prep_body,
        grid=grid,
        in_specs=[row, col, col],
        out_specs=[row, col, col],
        out_shape=[
            jax.ShapeDtypeStruct((2, NPAD, D), jnp.float32),
            jax.ShapeDtypeStruct((2, N, 1), jnp.float32),
            jax.ShapeDtypeStruct((2, N, 1), jnp.float32),
        ],
    )(feat_cat, do_cat, di_cat)


def _mm1_body(a_ref, nd_ref, so_ref, w_ref, b_ref, y_ref):
    x = a_ref[0] * nd_ref[0]
    y = jnp.dot(x, w_ref[...], preferred_element_type=jnp.float32)
    y = jnp.maximum(y + b_ref[...], 0.0)
    y_ref[0] = y * so_ref[0]


def _tc_mm1(agg_cat, nd_cat, ns_cat, w, b):
    grid = (2 * _G,)
    row = pl.BlockSpec((1, _BN, D), lambda i: (i // _G, i % _G, 0))
    col = pl.BlockSpec((1, _BN, 1), lambda i: (i // _G, i % _G, 0))
    full = pl.BlockSpec((D, D), lambda i: (0, 0))
    vec = pl.BlockSpec((1, D), lambda i: (0, 0))
    return pl.pallas_call(
        _mm1_body,
        grid=grid,
        in_specs=[row, col, col, full, vec],
        out_specs=row,
        out_shape=jax.ShapeDtypeStruct((2, NPAD, D), jnp.float32),
    )(agg_cat, nd_cat, ns_cat, w, b)


def _head_body(a_ref, nd_ref, w2_ref, b2_ref, wm1_ref, bm1_ref,
               wm2_ref, bm2_ref, c_ref, x_ref):
    x = a_ref[0] * nd_ref[0]
    c = jnp.dot(x, w2_ref[...], preferred_element_type=jnp.float32)
    c = jnp.maximum(c + b2_ref[...], 0.0)
    c_ref[...] = c
    t = jnp.dot(c, wm1_ref[...], preferred_element_type=jnp.float32)
    t = jnp.maximum(t + bm1_ref[...], 0.0)
    y = jnp.dot(t, wm2_ref[...], preferred_element_type=jnp.float32)
    x_ref[...] = jnp.maximum(y + bm2_ref[...], 0.0)


def _tc_head(g, agg_cat, nd_cat, w2, b2, wm1, bm1, wm2, bm2):
    grid = (_G,)
    part = pl.BlockSpec((1, _BN, D), lambda i, g=g: (g, i, 0))
    coln = pl.BlockSpec((1, _BN, 1), lambda i, g=g: (g, i, 0))
    row = pl.BlockSpec((_BN, D), lambda i: (i, 0))
    full = pl.BlockSpec((D, D), lambda i: (0, 0))
    vec = pl.BlockSpec((1, D), lambda i: (0, 0))
    return pl.pallas_call(
        _head_body,
        grid=grid,
        in_specs=[part, coln, full, vec, full, vec, full, vec],
        out_specs=[row, row],
        out_shape=[
            jax.ShapeDtypeStruct((N, D), jnp.float32),
            jax.ShapeDtypeStruct((N, D), jnp.float32),
        ],
    )(agg_cat, nd_cat, w2, b2, wm1, bm1, wm2, bm2)


def _pad_edges(idx, pad_value):
    pad = jnp.full((EPAD - E,), pad_value, jnp.int32)
    return jnp.concatenate([idx, pad]).reshape(R2D, CH)


def kernel(feat1, feat2, edge_index1, edge_index2, W1, b1, W2, b2,
           Wm1, bm1, Wm2, bm2):
    s1, d1 = edge_index1[0], edge_index1[1]
    s2, d2 = edge_index2[0], edge_index2[1]

    ones_c = jnp.ones((CH,), jnp.float32)
    zeros_deg = jnp.zeros((DPT,), jnp.float32)
    zrows = jnp.zeros((RPT, D), jnp.float32)

    dego1, degi1, dego2, degi2 = _sc_degrees(
        _pad_edges(s1, N), _pad_edges(d1, N),
        _pad_edges(s2, N), _pad_edges(d2, N),
        ones_c, zeros_deg)

    src_cat = jnp.concatenate([s1, s2 + NPAD])
    dst_cat = jnp.concatenate([d1, d2])

    feat_cat = jnp.stack([feat1, feat2])
    do_cat = jnp.stack([dego1[:N] + dego1[NPAD:NPAD + N],
                        dego2[:N] + dego2[NPAD:NPAD + N]]).reshape(2, N, 1)
    di_cat = jnp.stack([degi1[:N] + degi1[NPAD:NPAD + N],
                        degi2[:N] + degi2[NPAD:NPAD + N]]).reshape(2, N, 1)

    h0_cat, ns_cat, nd_cat = _tc_prep(feat_cat, do_cat, di_cat)

    agg1 = _sc_spmm(h0_cat.reshape(2 * NPAD, D), src_cat, dst_cat, zrows)
    h1_cat = _tc_mm1(agg1, nd_cat, ns_cat, W1, b1.reshape(1, D))
    agg2 = _sc_spmm(h1_cat.reshape(2 * NPAD, D), src_cat, dst_cat, zrows)

    c1, x21 = _tc_head(0, agg2, nd_cat, W2, b2.reshape(1, D),
                       Wm1, bm1.reshape(1, D), Wm2, bm2.reshape(1, D))
    c2, x22 = _tc_head(1, agg2, nd_cat, W2, b2.reshape(1, D),
                       Wm1, bm1.reshape(1, D), Wm2, bm2.reshape(1, D))
    return (c1, c2, x21, x22)


# speedup vs baseline: 1.9262x; 1.1776x over previous
import functools

import jax
import jax.numpy as jnp
from jax import lax
from jax.experimental import pallas as pl
from jax.experimental.pallas import tpu as pltpu
from jax.experimental.pallas import tpu_sc as plsc

N = 10000
E = 320000
D = 128

NC = 2
NS = 16
NW = NC * NS
CH = 128
R2D = 2560
EPAD = R2D * CH
RPC = R2D // NW
CHS = 80
ESW = E // NS
NCHS = ESW // CHS
NPAD = 10240
RPT = NPAD // NS
DPT = NPAD // NS

_sc_mesh = plsc.VectorSubcoreMesh(core_axis_name="c", subcore_axis_name="s")


@functools.partial(
    pl.kernel,
    out_type=[jax.ShapeDtypeStruct((NC * NPAD,), jnp.float32)] * 4,
    mesh=_sc_mesh,
    scratch_types=[
        pltpu.VMEM((RPC, CH), jnp.int32),
        pltpu.VMEM((RPC, CH), jnp.int32),
        pltpu.VMEM((RPC, CH), jnp.int32),
        pltpu.VMEM((RPC, CH), jnp.int32),
        pltpu.VMEM((CH,), jnp.float32),
        pltpu.VMEM_SHARED((NPAD,), jnp.float32),
        pltpu.VMEM_SHARED((NPAD,), jnp.float32),
        pltpu.VMEM_SHARED((NPAD,), jnp.float32),
        pltpu.VMEM_SHARED((NPAD,), jnp.float32),
        pltpu.SemaphoreType.DMA,
        pltpu.SemaphoreType.DMA,
        pltpu.SemaphoreType.DMA,
        pltpu.SemaphoreType.DMA,
    ],
)
def _sc_degrees(s1_hbm, d1_hbm, s2_hbm, d2_hbm, ones_hbm, zeros_hbm,
                o1_hbm, i1_hbm, o2_hbm, i2_hbm,
                s1_v, d1_v, s2_v, d2_v, ones_v,
                o1_sh, i1_sh, o2_sh, i2_sh,
                sem0, sem1, sem2, sem3):
    cid = lax.axis_index("c")
    sid = lax.axis_index("s")
    wid = cid * NS + sid
    idx_v = (s1_v, d1_v, s2_v, d2_v)
    idx_hbm = (s1_hbm, d1_hbm, s2_hbm, d2_hbm)
    sh = (o1_sh, i1_sh, o2_sh, i2_sh)
    out = (o1_hbm, i1_hbm, o2_hbm, i2_hbm)
    sems = (sem0, sem1, sem2, sem3)

    pltpu.sync_copy(ones_hbm, ones_v)
    for k in range(4):
        pltpu.sync_copy(idx_hbm[k].at[pl.ds(wid * RPC, RPC)], idx_v[k])
        pltpu.sync_copy(zeros_hbm, sh[k].at[pl.ds(sid * DPT, DPT)])
    plsc.subcore_barrier()

    for k in range(4):
        pltpu.async_copy(ones_v, sh[k].at[idx_v[k].at[0]], sems[k], add=True)

    def body(j, carry):
        for k in range(4):
            pltpu.make_async_copy(ones_v, sh[k].at[idx_v[k].at[j - 1]],
                                  sems[k]).wait()
            pltpu.async_copy(ones_v, sh[k].at[idx_v[k].at[j]], sems[k],
                             add=True)
        return carry

    lax.fori_loop(1, RPC, body, 0)
    for k in range(4):
        pltpu.make_async_copy(ones_v, sh[k].at[idx_v[k].at[RPC - 1]],
                              sems[k]).wait()
    plsc.subcore_barrier()
    for k in range(4):
        pltpu.sync_copy(sh[k].at[pl.ds(sid * DPT, DPT)],
                        out[k].at[pl.ds(cid * NPAD + sid * DPT, DPT)])


@functools.partial(
    pl.kernel,
    out_type=jax.ShapeDtypeStruct((NC, NPAD, D), jnp.float32),
    mesh=_sc_mesh,
    scratch_types=[
        pltpu.VMEM((ESW,), jnp.int32),
        pltpu.VMEM((CHS,), jnp.int32),
        pltpu.VMEM((CHS, D), jnp.float32),
        pltpu.VMEM_SHARED((NPAD, D), jnp.float32),
        pltpu.SemaphoreType.DMA,
    ],
)
def _sc_spmm(h_hbm, src_hbm, dst_hbm, zrows_hbm, out_hbm,
             sidx_v, didx_v, rows_v, acc_sh, sem):
    cid = lax.axis_index("c")
    sid = lax.axis_index("s")
    wid = cid * NS + sid
    base = wid * ESW

    pltpu.sync_copy(zrows_hbm, acc_sh.at[pl.ds(sid * RPT, RPT)])
    pltpu.sync_copy(src_hbm.at[pl.ds(base, ESW)], sidx_v)
    plsc.subcore_barrier()

    def body(i, carry):
        pltpu.sync_copy(dst_hbm.at[pl.ds(base + i * CHS, CHS)], didx_v)
        pltpu.async_copy(h_hbm.at[sidx_v.at[pl.ds(i * CHS, CHS)]], rows_v,
                         sem).wait()
        pltpu.sync_copy(rows_v, acc_sh.at[didx_v], add=True)
        return carry

    lax.fori_loop(0, NCHS, body, 0)
    plsc.subcore_barrier()
    pltpu.sync_copy(acc_sh.at[pl.ds(sid * RPT, RPT)],
                    out_hbm.at[cid, pl.ds(sid * RPT, RPT)])


_BN = 2000
_G = N // _BN


def _prep_body(feat_ref, do_ref, di_ref, h0_ref, ns_ref, nd_ref):
    dego = do_ref[0]
    degi = di_ref[0]
    ns = lax.rsqrt(jnp.where(dego > 0, dego, 1.0))
    nd = lax.rsqrt(jnp.where(degi > 0, degi, 1.0))
    ns_ref[0] = ns
    nd_ref[0] = nd
    h0_ref[0] = feat_ref[0] * ns


def _tc_prep(feat_cat, do_cat, di_cat):
    grid = (2 * _G,)
    row = pl.BlockSpec((1, _BN, D), lambda i: (i // _G, i % _G, 0))
    col = pl.BlockSpec((1, _BN, 1), lambda i: (i // _G, i % _G, 0))
    return pl.pallas_call(
        _prep_body,
        grid=grid,
        in_specs=[row, col, col],
        out_specs=[row, col, col],
        out_shape=[
            jax.ShapeDtypeStruct((2, NPAD, D), jnp.float32),
            jax.ShapeDtypeStruct((2, N, 1), jnp.float32),
            jax.ShapeDtypeStruct((2, N, 1), jnp.float32),
        ],
    )(feat_cat, do_cat, di_cat)


def _mm1_body(a_ref, nd_ref, so_ref, w_ref, b_ref, y_ref):
    x = a_ref[0] * nd_ref[0]
    y = jnp.dot(x, w_ref[...], preferred_element_type=jnp.float32)
    y = jnp.maximum(y + b_ref[...], 0.0)
    y_ref[0] = y * so_ref[0]


def _tc_mm1(agg_cat, nd_cat, ns_cat, w, b):
    grid = (2 * _G,)
    row = pl.BlockSpec((1, _BN, D), lambda i: (i // _G, i % _G, 0))
    col = pl.BlockSpec((1, _BN, 1), lambda i: (i // _G, i % _G, 0))
    full = pl.BlockSpec((D, D), lambda i: (0, 0))
    vec = pl.BlockSpec((1, D), lambda i: (0, 0))
    return pl.pallas_call(
        _mm1_body,
        grid=grid,
        in_specs=[row, col, col, full, vec],
        out_specs=row,
        out_shape=jax.ShapeDtypeStruct((2, NPAD, D), jnp.float32),
    )(agg_cat, nd_cat, ns_cat, w, b)


def _head_body(a_ref, nd_ref, w2_ref, b2_ref, wm1_ref, bm1_ref,
               wm2_ref, bm2_ref, c_ref, x_ref):
    x = a_ref[0] * nd_ref[0]
    c = jnp.dot(x, w2_ref[...], preferred_element_type=jnp.float32)
    c = jnp.maximum(c + b2_ref[...], 0.0)
    c_ref[...] = c
    t = jnp.dot(c, wm1_ref[...], preferred_element_type=jnp.float32)
    t = jnp.maximum(t + bm1_ref[...], 0.0)
    y = jnp.dot(t, wm2_ref[...], preferred_element_type=jnp.float32)
    x_ref[...] = jnp.maximum(y + bm2_ref[...], 0.0)


def _tc_head(g, agg_cat, nd_cat, w2, b2, wm1, bm1, wm2, bm2):
    grid = (_G,)
    part = pl.BlockSpec((1, _BN, D), lambda i, g=g: (g, i, 0))
    coln = pl.BlockSpec((1, _BN, 1), lambda i, g=g: (g, i, 0))
    row = pl.BlockSpec((_BN, D), lambda i: (i, 0))
    full = pl.BlockSpec((D, D), lambda i: (0, 0))
    vec = pl.BlockSpec((1, D), lambda i: (0, 0))
    return pl.pallas_call(
        _head_body,
        grid=grid,
        in_specs=[part, coln, full, vec, full, vec, full, vec],
        out_specs=[row, row],
        out_shape=[
            jax.ShapeDtypeStruct((N, D), jnp.float32),
            jax.ShapeDtypeStruct((N, D), jnp.float32),
        ],
    )(agg_cat, nd_cat, w2, b2, wm1, bm1, wm2, bm2)


def _pad_edges(idx, pad_value):
    pad = jnp.full((EPAD - E,), pad_value, jnp.int32)
    return jnp.concatenate([idx, pad]).reshape(R2D, CH)


def kernel(feat1, feat2, edge_index1, edge_index2, W1, b1, W2, b2,
           Wm1, bm1, Wm2, bm2):
    s1, d1 = edge_index1[0], edge_index1[1]
    s2, d2 = edge_index2[0], edge_index2[1]

    ones_c = jnp.ones((CH,), jnp.float32)
    zeros_deg = jnp.zeros((DPT,), jnp.float32)
    zrows = jnp.zeros((RPT, D), jnp.float32)

    dego1, degi1, dego2, degi2 = _sc_degrees(
        _pad_edges(s1, N), _pad_edges(d1, N),
        _pad_edges(s2, N), _pad_edges(d2, N),
        ones_c, zeros_deg)

    src_cat = jnp.concatenate([s1, s2 + NPAD])
    dst_cat = jnp.concatenate([d1, d2])

    feat_cat = jnp.stack([feat1, feat2])
    do_cat = jnp.stack([dego1[:N] + dego1[NPAD:NPAD + N],
                        dego2[:N] + dego2[NPAD:NPAD + N]]).reshape(2, N, 1)
    di_cat = jnp.stack([degi1[:N] + degi1[NPAD:NPAD + N],
                        degi2[:N] + degi2[NPAD:NPAD + N]]).reshape(2, N, 1)

    h0_cat, ns_cat, nd_cat = _tc_prep(feat_cat, do_cat, di_cat)

    agg1 = _sc_spmm(h0_cat.reshape(2 * NPAD, D), src_cat, dst_cat, zrows)
    h1_cat = _tc_mm1(agg1, nd_cat, ns_cat, W1, b1.reshape(1, D))
    agg2 = _sc_spmm(h1_cat.reshape(2 * NPAD, D), src_cat, dst_cat, zrows)

    c1, x21 = _tc_head(0, agg2, nd_cat, W2, b2.reshape(1, D),
                       Wm1, bm1.reshape(1, D), Wm2, bm2.reshape(1, D))
    c2, x22 = _tc_head(1, agg2, nd_cat, W2, b2.reshape(1, D),
                       Wm1, bm1.reshape(1, D), Wm2, bm2.reshape(1, D))
    return (c1, c2, x21, x22)
